# baseline (speedup 1.0000x reference)
import functools

import jax
import jax.numpy as jnp
from jax import lax
from jax.experimental import pallas as pl
from jax.experimental.pallas import tpu as pltpu
from jax.experimental.pallas import tpu_sc as plsc

_NC = 2
_NS = 16
_NW = _NC * _NS
_LANES = 16
_G = 64


@functools.lru_cache(maxsize=None)
def _sc_message_fn(N, E, D):
    assert E % _NW == 0
    epw = E // _NW
    C = 80
    assert epw % C == 0
    nch = epw // C
    assert nch % 2 == 1
    Np = -(-N // (_NS * 64)) * (_NS * 64)
    rps = Np // _NS
    RB = 64
    assert rps % RB == 0 and RB % 8 == 0 and C >= RB
    nrb = rps // RB

    mesh = plsc.VectorSubcoreMesh(core_axis_name="c", subcore_axis_name="s",
                                  num_cores=_NC, num_subcores=_NS)

    def body(h_hbm, src_hbm, dst_hbm, attr_hbm, out_hbm,
             src_c0, src_c1, dst_c0, dst_c1, abuf0, abuf1, gbuf0, gbuf1,
             sx0, sx1, sio0, sio1, ss0, ss1, agg_sh):
        src_c = (src_c0, src_c1)
        dst_c = (dst_c0, dst_c1)
        abuf = (abuf0, abuf1)
        gbuf = (gbuf0, gbuf1)
        sx = (sx0, sx1)
        sio = (sio0, sio1)
        ss = (ss0, ss1)
        c = lax.axis_index("c")
        s = lax.axis_index("s")
        wid = c * _NS + s

        def zrow(r, carry):
            for j in range(D // _LANES):
                abuf0[r, pl.ds(j * _LANES, _LANES)] = jnp.zeros(
                    (_LANES,), jnp.float32)
            return carry
        lax.fori_loop(0, RB, zrow, 0)
        for i in range(nrb):
            pltpu.async_copy(abuf0.at[pl.ds(0, RB)],
                             agg_sh.at[pl.ds(s * rps + i * RB, RB)], sx0)
        for i in range(nrb):
            pltpu.make_async_copy(
                abuf0.at[pl.ds(0, RB)],
                agg_sh.at[pl.ds(s * rps + i * RB, RB)], sx0).wait()

        plsc.subcore_barrier()

        def issue_src(k, b):
            pltpu.async_copy(src_hbm.at[wid, k], src_c[b], sx[b])

        def wait_src(k, b):
            pltpu.make_async_copy(src_hbm.at[wid, k], src_c[b], sx[b]).wait()

        def issue_in(k, b):
            pltpu.async_copy(h_hbm.at[src_c[b].at[0]], gbuf[b], sio[b])
            pltpu.async_copy(dst_hbm.at[wid, k], dst_c[b], sio[b])
            pltpu.async_copy(attr_hbm.at[wid, k], abuf[b], sio[b])

        def wait_in(k, b):
            pltpu.make_async_copy(h_hbm.at[src_c[b].at[0]], gbuf[b],
                                  sio[b]).wait()
            pltpu.make_async_copy(dst_hbm.at[wid, k], dst_c[b],
                                  sio[b]).wait()
            pltpu.make_async_copy(attr_hbm.at[wid, k], abuf[b],
                                  sio[b]).wait()

        def issue_scat(b):
            pltpu.async_copy(abuf[b], agg_sh.at[dst_c[b].at[0]], ss[b],
                             add=True)

        def wait_scat(b):
            pltpu.make_async_copy(abuf[b], agg_sh.at[dst_c[b].at[0]],
                                  ss[b]).wait()

        issue_src(0, 0)
        issue_src(1, 1)
        wait_src(0, 0)
        issue_in(0, 0)

        def step(t, carry):
            for u in range(2):
                k = 2 * t + u
                b, bp = u, 1 - u

                @pl.when(k < nch)
                def _visit():
                    @pl.when(k >= 1)
                    def _():
                        wait_scat(bp)

                    @pl.when(k + 1 < nch)
                    def _():
                        wait_src(k + 1, bp)
                        issue_in(k + 1, bp)

                    wait_in(k, b)

                    @pl.when(k + 2 < nch)
                    def _():
                        issue_src(k + 2, b)

                    def relu_row(r8, c2):
                        for u2 in range(8):
                            r = r8 * 8 + u2
                            for j in range(D // _LANES):
                                sl = pl.ds(j * _LANES, _LANES)
                                ab = abuf[b]
                                gb = gbuf[b]
                                ab[r, sl] = jnp.maximum(
                                    ab[r, sl] + gb[r, sl], 0.0)
                        return c2
                    lax.fori_loop(0, C // 8, relu_row, 0)

                    issue_scat(b)
            return carry
        lax.fori_loop(0, (nch + 2) // 2, step, 0)
        wait_scat((nch - 1) % 2)

        plsc.subcore_barrier()
        for i in range(nrb):
            sl = pl.ds(s * rps + i * RB, RB)
            pltpu.async_copy(agg_sh.at[sl], out_hbm.at[c, sl], sx0)
        for i in range(nrb):
            sl = pl.ds(s * rps + i * RB, RB)
            pltpu.make_async_copy(agg_sh.at[sl], out_hbm.at[c, sl],
                                  sx0).wait()

    fn = pl.kernel(
        body,
        out_type=jax.ShapeDtypeStruct((_NC, Np, D), jnp.float32),
        mesh=mesh,
        scratch_types=[
            pltpu.VMEM((1, C), jnp.int32),
            pltpu.VMEM((1, C), jnp.int32),
            pltpu.VMEM((1, C), jnp.int32),
            pltpu.VMEM((1, C), jnp.int32),
            pltpu.VMEM((C, D), jnp.float32),
            pltpu.VMEM((C, D), jnp.float32),
            pltpu.VMEM((C, D), jnp.float32),
            pltpu.VMEM((C, D), jnp.float32),
            pltpu.SemaphoreType.DMA,
            pltpu.SemaphoreType.DMA,
            pltpu.SemaphoreType.DMA,
            pltpu.SemaphoreType.DMA,
            pltpu.SemaphoreType.DMA,
            pltpu.SemaphoreType.DMA,
            pltpu.VMEM_SHARED((Np, D), jnp.float32),
        ],
    )
    return fn, C, nch


def _mlp_body(h_ref, agg_ref, w1_ref, b1_ref, w2_ref, b2_ref, batch_ref,
              hout_ref, pool_ref):
    n = h_ref.shape[0]
    a = agg_ref[...]
    z = h_ref[...] + a[0, :n] + a[1, :n]
    t = jnp.maximum(
        jnp.dot(z, w1_ref[...], preferred_element_type=jnp.float32)
        + b1_ref[...], 0.0)
    y = (jnp.dot(t, w2_ref[...], preferred_element_type=jnp.float32)
         + b2_ref[...])
    hn = jnp.maximum(y, 0.0)
    hout_ref[...] = hn
    onehot = (lax.broadcasted_iota(jnp.int32, (_G, n), 0)
              == batch_ref[...]).astype(jnp.float32)
    pool_ref[...] = jnp.dot(onehot, hn, preferred_element_type=jnp.float32)


@functools.lru_cache(maxsize=None)
def _mlp_fn(N, D):
    return pl.pallas_call(
        _mlp_body,
        out_shape=(jax.ShapeDtypeStruct((N, D), jnp.float32),
                   jax.ShapeDtypeStruct((_G, D), jnp.float32)),
    )


def kernel(x, edge_index, edge_attr, batch,
           W1_0, b1_0, W2_0, b2_0,
           W1_1, b1_1, W2_1, b2_1,
           W1_2, b1_2, W2_2, b2_2):
    N, D = x.shape
    E = edge_attr.shape[0]
    params = [(W1_0, b1_0, W2_0, b2_0),
              (W1_1, b1_1, W2_1, b2_1),
              (W1_2, b1_2, W2_2, b2_2)]

    sc_fn, C, nch = _sc_message_fn(N, E, D)
    mlp = _mlp_fn(N, D)

    src4 = edge_index[0].reshape(_NW, nch, 1, C)
    dst4 = edge_index[1].reshape(_NW, nch, 1, C)
    attr4 = edge_attr.reshape(_NW, nch, C, D)
    batch2 = batch.reshape(1, N)

    h = x
    xs, pools = [], []
    for (W1, b1, W2, b2) in params:
        agg = sc_fn(h, src4, dst4, attr4)
        h, pool = mlp(h, agg, W1, b1.reshape(1, D), W2, b2.reshape(1, D),
                      batch2)
        xs.append(h)
        pools.append(pool)
    return jnp.concatenate(pools, axis=1), jnp.concatenate(xs, axis=1)


# speedup vs baseline: 1.0156x; 1.0156x over previous
import functools

import jax
import jax.numpy as jnp
from jax import lax
from jax.experimental import pallas as pl
from jax.experimental.pallas import tpu as pltpu
from jax.experimental.pallas import tpu_sc as plsc

_NC = 2
_NS = 16
_NW = _NC * _NS
_LANES = 16
_G = 64


@functools.lru_cache(maxsize=None)
def _sc_message_fn(N, E, D):
    assert E % _NW == 0
    epw = E // _NW
    C = 80
    assert epw % C == 0
    nch = epw // C
    Np = -(-N // (_NS * 8)) * (_NS * 8)
    rps = Np // _NS
    RB = (min(C, rps) // 8) * 8
    row_blocks = []
    off = 0
    while off < rps:
        w = min(RB, rps - off)
        row_blocks.append((off, w))
        off += w

    mesh = plsc.VectorSubcoreMesh(core_axis_name="c", subcore_axis_name="s",
                                  num_cores=_NC, num_subcores=_NS)

    def body(h_hbm, src_hbm, dst_hbm, attr_hbm, out_hbm,
             src_c0, src_c1, dst_c0, dst_c1, abuf0, abuf1, gbuf0, gbuf1,
             sx0, sx1, sio0, sio1, ss0, ss1, agg_sh):
        src_c = (src_c0, src_c1)
        dst_c = (dst_c0, dst_c1)
        abuf = (abuf0, abuf1)
        gbuf = (gbuf0, gbuf1)
        sx = (sx0, sx1)
        sio = (sio0, sio1)
        ss = (ss0, ss1)
        c = lax.axis_index("c")
        s = lax.axis_index("s")
        wid = c * _NS + s

        def zrow(r, carry):
            for j in range(D // _LANES):
                abuf0[r, pl.ds(j * _LANES, _LANES)] = jnp.zeros(
                    (_LANES,), jnp.float32)
            return carry
        lax.fori_loop(0, RB, zrow, 0)
        for off, w in row_blocks:
            pltpu.async_copy(abuf0.at[pl.ds(0, w)],
                             agg_sh.at[pl.ds(s * rps + off, w)], sx0)
        for off, w in row_blocks:
            pltpu.make_async_copy(
                abuf0.at[pl.ds(0, w)],
                agg_sh.at[pl.ds(s * rps + off, w)], sx0).wait()

        plsc.subcore_barrier()

        def issue_src(k, b):
            pltpu.async_copy(src_hbm.at[wid, k], src_c[b], sx[b])

        def wait_src(k, b):
            pltpu.make_async_copy(src_hbm.at[wid, k], src_c[b], sx[b]).wait()

        def issue_in(k, b):
            pltpu.async_copy(h_hbm.at[src_c[b].at[0]], gbuf[b], sio[b])
            pltpu.async_copy(dst_hbm.at[wid, k], dst_c[b], sio[b])
            pltpu.async_copy(attr_hbm.at[wid, k], abuf[b], sio[b])

        def wait_in(k, b):
            pltpu.make_async_copy(h_hbm.at[src_c[b].at[0]], gbuf[b],
                                  sio[b]).wait()
            pltpu.make_async_copy(dst_hbm.at[wid, k], dst_c[b],
                                  sio[b]).wait()
            pltpu.make_async_copy(attr_hbm.at[wid, k], abuf[b],
                                  sio[b]).wait()

        def issue_scat(b):
            pltpu.async_copy(abuf[b], agg_sh.at[dst_c[b].at[0]], ss[b],
                             add=True)

        def wait_scat(b):
            pltpu.make_async_copy(abuf[b], agg_sh.at[dst_c[b].at[0]],
                                  ss[b]).wait()

        issue_src(0, 0)
        issue_src(1, 1)
        wait_src(0, 0)
        issue_in(0, 0)

        def step(t, carry):
            for u in range(2):
                k = 2 * t + u
                b, bp = u, 1 - u

                @pl.when(k < nch)
                def _visit():
                    @pl.when(k >= 1)
                    def _():
                        wait_scat(bp)

                    @pl.when(k + 1 < nch)
                    def _():
                        wait_src(k + 1, bp)
                        issue_in(k + 1, bp)

                    wait_in(k, b)

                    @pl.when(k + 2 < nch)
                    def _():
                        issue_src(k + 2, b)

                    def relu_row(r8, c2):
                        for u2 in range(5):
                            r = r8 * 5 + u2
                            for j in range(D // _LANES):
                                sl = pl.ds(j * _LANES, _LANES)
                                ab = abuf[b]
                                gb = gbuf[b]
                                ab[r, sl] = jnp.maximum(
                                    ab[r, sl] + gb[r, sl], 0.0)
                        return c2
                    lax.fori_loop(0, C // 5, relu_row, 0)

                    issue_scat(b)
            return carry
        lax.fori_loop(0, (nch + 2) // 2, step, 0)
        wait_scat((nch - 1) % 2)

        plsc.subcore_barrier()
        for off, w in row_blocks:
            sl = pl.ds(s * rps + off, w)
            pltpu.async_copy(agg_sh.at[sl], out_hbm.at[c, sl], sx0)
        for off, w in row_blocks:
            sl = pl.ds(s * rps + off, w)
            pltpu.make_async_copy(agg_sh.at[sl], out_hbm.at[c, sl],
                                  sx0).wait()

    fn = pl.kernel(
        body,
        out_type=jax.ShapeDtypeStruct((_NC, Np, D), jnp.float32),
        mesh=mesh,
        scratch_types=[
            pltpu.VMEM((1, C), jnp.int32),
            pltpu.VMEM((1, C), jnp.int32),
            pltpu.VMEM((1, C), jnp.int32),
            pltpu.VMEM((1, C), jnp.int32),
            pltpu.VMEM((C, D), jnp.float32),
            pltpu.VMEM((C, D), jnp.float32),
            pltpu.VMEM((C, D), jnp.float32),
            pltpu.VMEM((C, D), jnp.float32),
            pltpu.SemaphoreType.DMA,
            pltpu.SemaphoreType.DMA,
            pltpu.SemaphoreType.DMA,
            pltpu.SemaphoreType.DMA,
            pltpu.SemaphoreType.DMA,
            pltpu.SemaphoreType.DMA,
            pltpu.VMEM_SHARED((Np, D), jnp.float32),
        ],
    )
    return fn, C, nch


def _mlp_body(h_ref, agg_ref, w1_ref, b1_ref, w2_ref, b2_ref, batch_ref,
              hout_ref, pool_ref):
    n = h_ref.shape[0]
    a = agg_ref[...]
    z = h_ref[...] + a[0, :n] + a[1, :n]
    t = jnp.maximum(
        jnp.dot(z, w1_ref[...], preferred_element_type=jnp.float32)
        + b1_ref[...], 0.0)
    y = (jnp.dot(t, w2_ref[...], preferred_element_type=jnp.float32)
         + b2_ref[...])
    hn = jnp.maximum(y, 0.0)
    hout_ref[...] = hn
    onehot = (lax.broadcasted_iota(jnp.int32, (_G, n), 0)
              == batch_ref[...]).astype(jnp.float32)
    pool_ref[...] = jnp.dot(onehot, hn, preferred_element_type=jnp.float32)


@functools.lru_cache(maxsize=None)
def _mlp_fn(N, D):
    return pl.pallas_call(
        _mlp_body,
        out_shape=(jax.ShapeDtypeStruct((N, D), jnp.float32),
                   jax.ShapeDtypeStruct((_G, D), jnp.float32)),
    )


def kernel(x, edge_index, edge_attr, batch,
           W1_0, b1_0, W2_0, b2_0,
           W1_1, b1_1, W2_1, b2_1,
           W1_2, b1_2, W2_2, b2_2):
    N, D = x.shape
    E = edge_attr.shape[0]
    params = [(W1_0, b1_0, W2_0, b2_0),
              (W1_1, b1_1, W2_1, b2_1),
              (W1_2, b1_2, W2_2, b2_2)]

    sc_fn, C, nch = _sc_message_fn(N, E, D)
    mlp = _mlp_fn(N, D)

    src4 = edge_index[0].reshape(_NW, nch, 1, C)
    dst4 = edge_index[1].reshape(_NW, nch, 1, C)
    attr4 = edge_attr.reshape(_NW, nch, C, D)
    batch2 = batch.reshape(1, N)

    h = x
    xs, pools = [], []
    for (W1, b1, W2, b2) in params:
        agg = sc_fn(h, src4, dst4, attr4)
        h, pool = mlp(h, agg, W1, b1.reshape(1, D), W2, b2.reshape(1, D),
                      batch2)
        xs.append(h)
        pools.append(pool)
    return jnp.concatenate(pools, axis=1), jnp.concatenate(xs, axis=1)


# speedup vs baseline: 1.0176x; 1.0019x over previous
import functools

import jax
import jax.numpy as jnp
from jax import lax
from jax.experimental import pallas as pl
from jax.experimental.pallas import tpu as pltpu
from jax.experimental.pallas import tpu_sc as plsc

_NC = 2
_NS = 16
_NW = _NC * _NS
_LANES = 16
_G = 64


@functools.lru_cache(maxsize=None)
def _sc_message_fn(N, E, D):
    assert E % _NW == 0
    epw = E // _NW
    C = 80
    assert epw % C == 0
    nch = epw // C
    Np = -(-N // (_NS * 8)) * (_NS * 8)
    rps = Np // _NS
    RB = (min(C, rps) // 8) * 8
    row_blocks = []
    off = 0
    while off < rps:
        w = min(RB, rps - off)
        row_blocks.append((off, w))
        off += w

    mesh = plsc.VectorSubcoreMesh(core_axis_name="c", subcore_axis_name="s",
                                  num_cores=_NC, num_subcores=_NS)

    def body(h_hbm, src_hbm, dst_hbm, attr_hbm, out_hbm,
             src_c0, src_c1, dst_c0, dst_c1, abuf0, abuf1, gbuf0, gbuf1,
             sx0, sx1, sio0, sio1, ss0, ss1, agg_sh):
        src_c = (src_c0, src_c1)
        dst_c = (dst_c0, dst_c1)
        abuf = (abuf0, abuf1)
        gbuf = (gbuf0, gbuf1)
        sx = (sx0, sx1)
        sio = (sio0, sio1)
        ss = (ss0, ss1)
        c = lax.axis_index("c")
        s = lax.axis_index("s")
        wid = c * _NS + s

        def zrow(r, carry):
            for j in range(D // _LANES):
                abuf0[r, pl.ds(j * _LANES, _LANES)] = jnp.zeros(
                    (_LANES,), jnp.float32)
            return carry
        lax.fori_loop(0, RB, zrow, 0)
        for off, w in row_blocks:
            pltpu.async_copy(abuf0.at[pl.ds(0, w)],
                             agg_sh.at[pl.ds(s * rps + off, w)], sx0)
        for off, w in row_blocks:
            pltpu.make_async_copy(
                abuf0.at[pl.ds(0, w)],
                agg_sh.at[pl.ds(s * rps + off, w)], sx0).wait()

        plsc.subcore_barrier()

        def issue_src(k, b):
            pltpu.async_copy(src_hbm.at[wid, k], src_c[b], sx[b])

        def wait_src(k, b):
            pltpu.make_async_copy(src_hbm.at[wid, k], src_c[b], sx[b]).wait()

        def issue_in(k, b):
            pltpu.async_copy(h_hbm.at[src_c[b].at[0]], gbuf[b], sio[b])
            pltpu.async_copy(dst_hbm.at[wid, k], dst_c[b], sio[b])
            pltpu.async_copy(attr_hbm.at[wid, k], abuf[b], sio[b])

        def wait_in(k, b):
            pltpu.make_async_copy(h_hbm.at[src_c[b].at[0]], gbuf[b],
                                  sio[b]).wait()
            pltpu.make_async_copy(dst_hbm.at[wid, k], dst_c[b],
                                  sio[b]).wait()
            pltpu.make_async_copy(attr_hbm.at[wid, k], abuf[b],
                                  sio[b]).wait()

        def issue_scat(b):
            pltpu.async_copy(abuf[b], agg_sh.at[dst_c[b].at[0]], ss[b],
                             add=True)

        def wait_scat(b):
            pltpu.make_async_copy(abuf[b], agg_sh.at[dst_c[b].at[0]],
                                  ss[b]).wait()

        issue_src(0, 0)
        issue_src(1, 1)
        wait_src(0, 0)
        issue_in(0, 0)

        def step(t, carry):
            for u in range(2):
                k = 2 * t + u
                b, bp = u, 1 - u

                @pl.when(k < nch)
                def _visit():
                    @pl.when(k >= 1)
                    def _():
                        wait_scat(bp)

                    @pl.when(k + 1 < nch)
                    def _():
                        wait_src(k + 1, bp)
                        issue_in(k + 1, bp)

                    wait_in(k, b)

                    @pl.when(k + 2 < nch)
                    def _():
                        issue_src(k + 2, b)

                    def relu_row(r8, c2):
                        for u2 in range(5):
                            r = r8 * 5 + u2
                            ab = abuf[b]
                            gb = gbuf[b]
                            for j in range(D // _LANES):
                                sl = pl.ds(j * _LANES, _LANES)
                                ab[r, sl] = jnp.maximum(
                                    ab[r, sl] + gb[r, sl], 0.0)
                        return c2
                    lax.fori_loop(0, C // 5, relu_row, 0)

                    issue_scat(b)
            return carry
        lax.fori_loop(0, (nch + 2) // 2, step, 0)
        wait_scat((nch - 1) % 2)

        plsc.subcore_barrier()
        for off, w in row_blocks:
            sl = pl.ds(s * rps + off, w)
            pltpu.async_copy(agg_sh.at[sl], out_hbm.at[c, sl], sx0)
        for off, w in row_blocks:
            sl = pl.ds(s * rps + off, w)
            pltpu.make_async_copy(agg_sh.at[sl], out_hbm.at[c, sl],
                                  sx0).wait()

    fn = pl.kernel(
        body,
        out_type=jax.ShapeDtypeStruct((_NC, Np, D), jnp.float32),
        mesh=mesh,
        scratch_types=[
            pltpu.VMEM((1, C), jnp.int32),
            pltpu.VMEM((1, C), jnp.int32),
            pltpu.VMEM((1, C), jnp.int32),
            pltpu.VMEM((1, C), jnp.int32),
            pltpu.VMEM((C, D), jnp.float32),
            pltpu.VMEM((C, D), jnp.float32),
            pltpu.VMEM((C, D), jnp.float32),
            pltpu.VMEM((C, D), jnp.float32),
            pltpu.SemaphoreType.DMA,
            pltpu.SemaphoreType.DMA,
            pltpu.SemaphoreType.DMA,
            pltpu.SemaphoreType.DMA,
            pltpu.SemaphoreType.DMA,
            pltpu.SemaphoreType.DMA,
            pltpu.VMEM_SHARED((Np, D), jnp.float32),
        ],
    )
    return fn, C, nch


def _mlp_body(h_ref, agg_ref, w1_ref, b1_ref, w2_ref, b2_ref, batch_ref,
              hout_ref, pool_ref):
    n = h_ref.shape[0]
    a = agg_ref[...]
    z = h_ref[...] + a[0, :n] + a[1, :n]
    t = jnp.maximum(
        jnp.dot(z, w1_ref[...], preferred_element_type=jnp.float32)
        + b1_ref[...], 0.0)
    y = (jnp.dot(t, w2_ref[...], preferred_element_type=jnp.float32)
         + b2_ref[...])
    hn = jnp.maximum(y, 0.0)
    hout_ref[...] = hn
    onehot = (lax.broadcasted_iota(jnp.int32, (_G, n), 0)
              == batch_ref[...]).astype(jnp.float32)
    pool_ref[...] = jnp.dot(onehot, hn, preferred_element_type=jnp.float32)


@functools.lru_cache(maxsize=None)
def _mlp_fn(N, D):
    return pl.pallas_call(
        _mlp_body,
        out_shape=(jax.ShapeDtypeStruct((N, D), jnp.float32),
                   jax.ShapeDtypeStruct((_G, D), jnp.float32)),
    )


def kernel(x, edge_index, edge_attr, batch,
           W1_0, b1_0, W2_0, b2_0,
           W1_1, b1_1, W2_1, b2_1,
           W1_2, b1_2, W2_2, b2_2):
    N, D = x.shape
    E = edge_attr.shape[0]
    params = [(W1_0, b1_0, W2_0, b2_0),
              (W1_1, b1_1, W2_1, b2_1),
              (W1_2, b1_2, W2_2, b2_2)]

    sc_fn, C, nch = _sc_message_fn(N, E, D)
    mlp = _mlp_fn(N, D)

    src4 = edge_index[0].reshape(_NW, nch, 1, C)
    dst4 = edge_index[1].reshape(_NW, nch, 1, C)
    attr4 = edge_attr.reshape(_NW, nch, C, D)
    batch2 = batch.reshape(1, N)

    h = x
    xs, pools = [], []
    for (W1, b1, W2, b2) in params:
        agg = sc_fn(h, src4, dst4, attr4)
        h, pool = mlp(h, agg, W1, b1.reshape(1, D), W2, b2.reshape(1, D),
                      batch2)
        xs.append(h)
        pools.append(pool)
    return jnp.concatenate(pools, axis=1), jnp.concatenate(xs, axis=1)
